# baseline (speedup 1.0000x reference)
import functools

import jax
import jax.numpy as jnp
from jax import lax
from jax.experimental import pallas as pl
from jax.experimental.pallas import tpu as pltpu
from jax.experimental.pallas import tpu_sc as plsc


def _sc_gather(table, idx, chunk=2000):
  B = idx.shape[0]
  D = table.shape[1]
  info = plsc.get_sparse_core_info()
  nw = info.num_cores * info.num_subcores
  b_per_w = B // nw
  assert B % nw == 0 and b_per_w % chunk == 0, (B, nw, chunk)
  n_iter = b_per_w // chunk
  mesh = plsc.VectorSubcoreMesh(core_axis_name="c", subcore_axis_name="s")

  @functools.partial(
      pl.kernel,
      out_type=jax.ShapeDtypeStruct((B, D), table.dtype),
      mesh=mesh,
      scratch_types=[
          pltpu.VMEM((chunk,), jnp.int32),
          pltpu.VMEM((chunk, D), table.dtype),
          pltpu.SemaphoreType.DMA,
      ],
      compiler_params=pltpu.CompilerParams(use_tc_tiling_on_sc=False),
  )
  def k(table_hbm, idx_hbm, out_hbm, idx_v, rows_v, sem):
    wid = lax.axis_index("s") * info.num_cores + lax.axis_index("c")
    base = wid * b_per_w

    def body(i, _):
      start = base + i * chunk
      pltpu.sync_copy(idx_hbm.at[pl.ds(start, chunk)], idx_v)
      pltpu.async_copy(table_hbm.at[idx_v], rows_v, sem).wait()
      pltpu.sync_copy(rows_v, out_hbm.at[pl.ds(start, chunk)])
      return 0

    lax.fori_loop(0, n_iter, body, 0)

  return k(table, idx)


def _stage_channels(src4, stage_flat, t0, nt, sem):
  ntr = src4.shape[0]
  width = nt * 128
  cps = []
  for tr in range(ntr):
    for r in range(8):
      c = tr * 8 + r
      for t in range(nt):
        cps.append(pltpu.async_copy(
            src4.at[tr, t0 + t, r, :],
            stage_flat.at[pl.ds(c * width + t * 128, 128)], sem))
  return cps


def _assemble_rows(stage_flat, rows_v, n_rows, width):
  base = lax.iota(jnp.int32, 16) * width
  offs = [jnp.full((16,), u, jnp.int32) for u in range(8)]
  eight = jnp.full((16,), 8, jnp.int32)

  def body8(g, vec):
    for u in range(8):
      rows_v[g * 8 + u] = plsc.load_gather(stage_flat, [vec + offs[u]])
    return vec + eight

  lax.fori_loop(0, n_rows // 8, body8, base)


def _phys4(x):
  r, d = x.shape
  assert d == 16 and r % 128 == 0
  return x.T.reshape(2, 8, r // 128, 128).swapaxes(1, 2)


def _sc_transpose4(data4, nt_chunk=10):
  ntr, nt_all, _, _ = data4.shape
  R = nt_all * 128
  D = ntr * 8
  info = plsc.get_sparse_core_info()
  nw = info.num_cores * info.num_subcores
  assert nt_all % nt_chunk == 0
  n_chunks = nt_all // nt_chunk
  rows = nt_chunk * 128
  mesh = plsc.VectorSubcoreMesh(core_axis_name="c", subcore_axis_name="s")

  @functools.partial(
      pl.kernel,
      out_type=jax.ShapeDtypeStruct((R, D), jnp.float32),
      mesh=mesh,
      scratch_types=[
          pltpu.VMEM((D * nt_chunk * 128,), jnp.float32),
          pltpu.VMEM((rows, D), jnp.float32),
          pltpu.SemaphoreType.DMA,
      ],
      compiler_params=pltpu.CompilerParams(use_tc_tiling_on_sc=False,
                                           needs_layout_passes=False),
  )
  def k(src_hbm, out_hbm, stage_v, rows_v, sem):
    wid = lax.axis_index("s") * info.num_cores + lax.axis_index("c")
    n_mine = (n_chunks - wid + nw - 1) // nw

    def body(i, _):
      cid = wid + i * nw
      t0 = cid * nt_chunk
      cps = _stage_channels(src_hbm, stage_v, t0, nt_chunk, sem)
      for cp in cps:
        cp.wait()
      _assemble_rows(stage_v, rows_v, rows, rows)
      pltpu.sync_copy(rows_v, out_hbm.at[pl.ds(t0 * 128, rows)])
      return 0

    lax.fori_loop(0, n_mine, body, 0)

  return k(data4)


def _sc_segsum(data4, seg, n_out, nt_chunk=4, zchunk=400):
  ntr, nt_all, _, _ = data4.shape
  H = nt_all * 128
  D = ntr * 8
  info = plsc.get_sparse_core_info()
  nc, ns = info.num_cores, info.num_subcores
  nw = nc * ns
  assert nt_all % nt_chunk == 0
  n_chunks = nt_all // nt_chunk
  chunk = nt_chunk * 128
  assert n_out % zchunk == 0 and zchunk % 8 == 0
  n_z = n_out // zchunk
  mesh = plsc.VectorSubcoreMesh(core_axis_name="c", subcore_axis_name="s")

  @functools.partial(
      pl.kernel,
      out_type=jax.ShapeDtypeStruct((nc * n_out, D), jnp.float32),
      mesh=mesh,
      scratch_types=[
          pltpu.VMEM((chunk,), jnp.int32),
          pltpu.VMEM((D * nt_chunk * 128,), jnp.float32),
          pltpu.VMEM((chunk, D), jnp.float32),
          pltpu.VMEM_SHARED((n_out, D), jnp.float32),
          pltpu.SemaphoreType.DMA,
      ],
      compiler_params=pltpu.CompilerParams(use_tc_tiling_on_sc=False,
                                           needs_layout_passes=False),
  )
  def k(data_hbm, seg_hbm, out_hbm, idx_v, stage_v, rows_v, acc_sp, sem):
    cid = lax.axis_index("c")
    sid = lax.axis_index("s")
    wid = sid * nc + cid

    def zrow(i, _):
      rows_v[i] = jnp.zeros((D,), jnp.float32)
      return 0

    lax.fori_loop(0, min(chunk, zchunk), zrow, 0)

    def zbody(c, _):
      pltpu.sync_copy(rows_v.at[pl.ds(0, zchunk)],
                      acc_sp.at[pl.ds(c * zchunk, zchunk)])
      return 0

    def zloop(t0):
      n_mine = (n_z - t0 + ns - 1) // ns
      lax.fori_loop(0, n_mine, lambda j, _: zbody(t0 + j * ns, _), 0)

    zloop(sid)
    plsc.subcore_barrier()

    n_mine = (n_chunks - wid + nw - 1) // nw

    def body(i, _):
      cidx = wid + i * nw
      t0 = cidx * nt_chunk
      pltpu.sync_copy(seg_hbm.at[pl.ds(t0 * 128, chunk)], idx_v)
      cps = _stage_channels(data_hbm, stage_v, t0, nt_chunk, sem)
      for cp in cps:
        cp.wait()
      _assemble_rows(stage_v, rows_v, chunk, chunk)
      pltpu.sync_copy(rows_v, acc_sp.at[idx_v], add=True)
      return 0

    lax.fori_loop(0, n_mine, body, 0)
    plsc.subcore_barrier()

    def wbody(c, _):
      o = c * zchunk
      pltpu.sync_copy(acc_sp.at[pl.ds(o, zchunk)],
                      out_hbm.at[pl.ds(cid * n_out + o, zchunk)])
      return 0

    def wloop(t0):
      n_mine = (n_z - t0 + ns - 1) // ns
      lax.fori_loop(0, n_mine, lambda j, _: wbody(t0 + j * ns, _), 0)

    wloop(sid)

  return k(data4, seg)


def _tc_main(nodes, gathered2, hyp_parts, Wq_t, bq_t, W_bd, bk_t, S,
             W1n, W1a, W1h, b1_eff, W2, b2, gamma, beta, nb=1000):
  n = nodes.shape[0]
  d_h = hyp_parts.shape[2]
  l2 = W2.shape[1]
  assert n % nb == 0

  def body(x_ref, g_ref, hp_ref, wqt_ref, bqt_ref, wbd_ref, bkt_ref, s_ref,
           w1n_ref, w1a_ref, w1h_ref, b1_ref, w2_ref, b2_ref,
           gamma_ref, beta_ref, o_ref):
    f32 = jnp.float32
    x = x_ref[...]
    qh_t = jnp.dot(x, wqt_ref[...], preferred_element_type=f32) + bqt_ref[...]
    kg = jnp.dot(g_ref[...], wbd_ref[...], preferred_element_type=f32)
    kg = kg + bkt_ref[...]
    s = qh_t * kg
    m = jnp.max(s, axis=-1, keepdims=True)
    w = jnp.exp(s - m)
    sel = s_ref[...]
    z = jnp.dot(w, sel, preferred_element_type=f32)
    att = jnp.dot(w * kg, sel, preferred_element_type=f32) / z
    hyp = hp_ref[0] + hp_ref[1]
    pre1 = (jnp.dot(x, w1n_ref[...], preferred_element_type=f32)
            + jnp.dot(att, w1a_ref[...], preferred_element_type=f32)
            + jnp.dot(hyp, w1h_ref[...], preferred_element_type=f32)
            + b1_ref[...])
    h1 = jnp.maximum(pre1, 0.0)
    h2 = jnp.dot(h1, w2_ref[...], preferred_element_type=f32)
    h2 = jnp.maximum(h2 + b2_ref[...], 0.0)
    mean = jnp.mean(h2, axis=-1, keepdims=True)
    var = jnp.mean((h2 - mean) * (h2 - mean), axis=-1, keepdims=True)
    o_ref[...] = ((h2 - mean) * lax.rsqrt(var + 1e-3) * gamma_ref[...]
                  + beta_ref[...])

  grid = (n // nb,)
  full = lambda shape: pl.BlockSpec(shape, lambda i: (0,) * len(shape))
  return pl.pallas_call(
      body,
      grid=grid,
      in_specs=[
          pl.BlockSpec((nb, nodes.shape[1]), lambda i: (i, 0)),
          pl.BlockSpec((nb, gathered2.shape[1]), lambda i: (i, 0)),
          pl.BlockSpec((2, nb, d_h), lambda i: (0, i, 0)),
          full(Wq_t.shape), full(bq_t.shape), full(W_bd.shape), full(bk_t.shape),
          full(S.shape),
          full(W1n.shape), full(W1a.shape), full(W1h.shape), full(b1_eff.shape),
          full(W2.shape), full(b2.shape), full(gamma.shape), full(beta.shape),
      ],
      out_specs=pl.BlockSpec((nb, l2), lambda i: (i, 0)),
      out_shape=jax.ShapeDtypeStruct((n, l2), jnp.float32),
  )(nodes, gathered2, hyp_parts, Wq_t, bq_t, W_bd, bk_t, S,
    W1n, W1a, W1h, b1_eff, W2, b2, gamma, beta)


def kernel(nodes, globals_, edges, edge_ind, hyper_feat, hyper_ind,
           Wq, bq, Wk, bk, Wc, bc, W1, b1, W2, b2, gamma, beta):
  n, d_feat = nodes.shape
  e, d_edge = edges.shape
  kk = edge_ind.shape[1]
  d_glob = globals_.shape[1]
  ah = Wc.shape[2]
  d_hyp = hyper_feat.shape[1]

  Wc1 = Wc[1]
  Wqc = Wq @ Wc1
  bqc = (bq @ Wc1 + bc)[None, :]
  Wkc = Wk @ Wc1
  bkc = (bk @ Wc1 + bc)[None, :]

  Wq_t = jnp.tile(Wqc, (1, kk))
  bq_t = jnp.tile(bqc, (1, kk))
  W_bd = jnp.kron(jnp.eye(kk, dtype=jnp.float32), Wkc)
  bk_t = jnp.tile(bkc, (1, kk))
  S = jnp.tile(jnp.eye(ah, dtype=jnp.float32), (kk, 1))

  W1n = W1[:d_feat]
  W1g = W1[d_feat:d_feat + d_glob]
  W1a = W1[d_feat + d_glob:d_feat + d_glob + ah]
  W1h = W1[d_feat + d_glob + ah:]
  b1_eff = (b1 + (globals_ @ W1g)[0])[None, :]

  table = _sc_transpose4(_phys4(edges))
  idx = edge_ind.astype(jnp.int32).reshape(-1)
  gathered2 = _sc_gather(table, idx).reshape(n, kk * d_edge)

  seg = hyper_ind.astype(jnp.int32)
  hyp_parts = _sc_segsum(_phys4(hyper_feat), seg, n).reshape(2, n, d_hyp)

  out = _tc_main(nodes, gathered2, hyp_parts, Wq_t, bq_t, W_bd, bk_t, S,
                 W1n, W1a, W1h, b1_eff, W2, b2[None, :],
                 gamma[None, :], beta[None, :])
  return out


# speedup vs baseline: 1.2240x; 1.2240x over previous
import functools

import jax
import jax.numpy as jnp
from jax import lax
from jax.experimental import pallas as pl
from jax.experimental.pallas import tpu as pltpu
from jax.experimental.pallas import tpu_sc as plsc


def _sc_gather(table, idx, chunk=2000):
  B = idx.shape[0]
  D = table.shape[1]
  info = plsc.get_sparse_core_info()
  nw = info.num_cores * info.num_subcores
  b_per_w = B // nw
  assert B % nw == 0 and b_per_w % chunk == 0, (B, nw, chunk)
  n_iter = b_per_w // chunk
  mesh = plsc.VectorSubcoreMesh(core_axis_name="c", subcore_axis_name="s")

  @functools.partial(
      pl.kernel,
      out_type=jax.ShapeDtypeStruct((B, D), table.dtype),
      mesh=mesh,
      scratch_types=[
          pltpu.VMEM((chunk,), jnp.int32),
          pltpu.VMEM((chunk, D), table.dtype),
          pltpu.SemaphoreType.DMA,
      ],
      compiler_params=pltpu.CompilerParams(use_tc_tiling_on_sc=False),
  )
  def k(table_hbm, idx_hbm, out_hbm, idx_v, rows_v, sem):
    wid = lax.axis_index("s") * info.num_cores + lax.axis_index("c")
    base = wid * b_per_w

    def body(i, _):
      start = base + i * chunk
      pltpu.sync_copy(idx_hbm.at[pl.ds(start, chunk)], idx_v)
      pltpu.async_copy(table_hbm.at[idx_v], rows_v, sem).wait()
      pltpu.sync_copy(rows_v, out_hbm.at[pl.ds(start, chunk)])
      return 0

    lax.fori_loop(0, n_iter, body, 0)

  return k(table, idx)


def _sc_segsum(data, seg, n_out, chunk=1000, zchunk=1000):
  H, D = data.shape
  info = plsc.get_sparse_core_info()
  nc, ns = info.num_cores, info.num_subcores
  nw = nc * ns
  h_per_w = H // nw
  assert H % nw == 0 and h_per_w % chunk == 0
  assert n_out % zchunk == 0 and zchunk % 8 == 0
  n_iter = h_per_w // chunk
  n_z = n_out // zchunk
  mesh = plsc.VectorSubcoreMesh(core_axis_name="c", subcore_axis_name="s")

  @functools.partial(
      pl.kernel,
      out_type=jax.ShapeDtypeStruct((nc * n_out, D), jnp.float32),
      mesh=mesh,
      scratch_types=[
          pltpu.VMEM((chunk,), jnp.int32),
          pltpu.VMEM((chunk, D), jnp.float32),
          pltpu.VMEM_SHARED((n_out, D), jnp.float32),
          pltpu.SemaphoreType.DMA,
      ],
      compiler_params=pltpu.CompilerParams(use_tc_tiling_on_sc=False),
  )
  def k(data_hbm, seg_hbm, out_hbm, idx_v, rows_v, acc_sp, sem):
    cid = lax.axis_index("c")
    sid = lax.axis_index("s")
    wid = sid * nc + cid

    def zrow(i, _):
      rows_v[i] = jnp.zeros((D,), jnp.float32)
      return 0

    lax.fori_loop(0, chunk, zrow, 0)

    def zbody(c, _):
      pltpu.sync_copy(rows_v.at[pl.ds(0, zchunk)],
                      acc_sp.at[pl.ds(c * zchunk, zchunk)])
      return 0

    def zloop(t0):
      n_mine = (n_z - t0 + ns - 1) // ns
      lax.fori_loop(0, n_mine, lambda j, _: zbody(t0 + j * ns, _), 0)

    zloop(sid)
    plsc.subcore_barrier()

    base = wid * h_per_w

    def body(i, _):
      start = base + i * chunk
      pltpu.sync_copy(seg_hbm.at[pl.ds(start, chunk)], idx_v)
      pltpu.sync_copy(data_hbm.at[pl.ds(start, chunk)], rows_v)
      pltpu.sync_copy(rows_v, acc_sp.at[idx_v], add=True)
      return 0

    lax.fori_loop(0, n_iter, body, 0)
    plsc.subcore_barrier()

    def wbody(c, _):
      o = c * zchunk
      pltpu.sync_copy(acc_sp.at[pl.ds(o, zchunk)],
                      out_hbm.at[pl.ds(cid * n_out + o, zchunk)])
      return 0

    def wloop(t0):
      n_mine = (n_z - t0 + ns - 1) // ns
      lax.fori_loop(0, n_mine, lambda j, _: wbody(t0 + j * ns, _), 0)

    wloop(sid)

  return k(data, seg)


def _tc_main(nodes, gathered2, hyp_parts, Wq_t, bq_t, W_bd, bk_t, S,
             W1n, W1a, W1h, b1_eff, W2, b2, gamma, beta, nb=1000):
  n = nodes.shape[0]
  d_h = hyp_parts.shape[2]
  l2 = W2.shape[1]
  assert n % nb == 0

  def body(x_ref, g_ref, hp_ref, wqt_ref, bqt_ref, wbd_ref, bkt_ref, s_ref,
           w1n_ref, w1a_ref, w1h_ref, b1_ref, w2_ref, b2_ref,
           gamma_ref, beta_ref, o_ref):
    f32 = jnp.float32
    x = x_ref[...]
    qh_t = jnp.dot(x, wqt_ref[...], preferred_element_type=f32) + bqt_ref[...]
    kg = jnp.dot(g_ref[...], wbd_ref[...], preferred_element_type=f32)
    kg = kg + bkt_ref[...]
    s = qh_t * kg
    m = jnp.max(s, axis=-1, keepdims=True)
    w = jnp.exp(s - m)
    sel = s_ref[...]
    z = jnp.dot(w, sel, preferred_element_type=f32)
    att = jnp.dot(w * kg, sel, preferred_element_type=f32) / z
    hyp = hp_ref[0] + hp_ref[1]
    pre1 = (jnp.dot(x, w1n_ref[...], preferred_element_type=f32)
            + jnp.dot(att, w1a_ref[...], preferred_element_type=f32)
            + jnp.dot(hyp, w1h_ref[...], preferred_element_type=f32)
            + b1_ref[...])
    h1 = jnp.maximum(pre1, 0.0)
    h2 = jnp.dot(h1, w2_ref[...], preferred_element_type=f32)
    h2 = jnp.maximum(h2 + b2_ref[...], 0.0)
    mean = jnp.mean(h2, axis=-1, keepdims=True)
    var = jnp.mean((h2 - mean) * (h2 - mean), axis=-1, keepdims=True)
    o_ref[...] = ((h2 - mean) * lax.rsqrt(var + 1e-3) * gamma_ref[...]
                  + beta_ref[...])

  grid = (n // nb,)
  full = lambda shape: pl.BlockSpec(shape, lambda i: (0,) * len(shape))
  return pl.pallas_call(
      body,
      grid=grid,
      in_specs=[
          pl.BlockSpec((nb, nodes.shape[1]), lambda i: (i, 0)),
          pl.BlockSpec((nb, gathered2.shape[1]), lambda i: (i, 0)),
          pl.BlockSpec((2, nb, d_h), lambda i: (0, i, 0)),
          full(Wq_t.shape), full(bq_t.shape), full(W_bd.shape), full(bk_t.shape),
          full(S.shape),
          full(W1n.shape), full(W1a.shape), full(W1h.shape), full(b1_eff.shape),
          full(W2.shape), full(b2.shape), full(gamma.shape), full(beta.shape),
      ],
      out_specs=pl.BlockSpec((nb, l2), lambda i: (i, 0)),
      out_shape=jax.ShapeDtypeStruct((n, l2), jnp.float32),
  )(nodes, gathered2, hyp_parts, Wq_t, bq_t, W_bd, bk_t, S,
    W1n, W1a, W1h, b1_eff, W2, b2, gamma, beta)


def kernel(nodes, globals_, edges, edge_ind, hyper_feat, hyper_ind,
           Wq, bq, Wk, bk, Wc, bc, W1, b1, W2, b2, gamma, beta):
  n, d_feat = nodes.shape
  e, d_edge = edges.shape
  kk = edge_ind.shape[1]
  d_glob = globals_.shape[1]
  ah = Wc.shape[2]
  d_hyp = hyper_feat.shape[1]

  Wc1 = Wc[1]
  Wqc = Wq @ Wc1
  bqc = (bq @ Wc1 + bc)[None, :]
  Wkc = Wk @ Wc1
  bkc = (bk @ Wc1 + bc)[None, :]

  Wq_t = jnp.tile(Wqc, (1, kk))
  bq_t = jnp.tile(bqc, (1, kk))
  W_bd = jnp.kron(jnp.eye(kk, dtype=jnp.float32), Wkc)
  bk_t = jnp.tile(bkc, (1, kk))
  S = jnp.tile(jnp.eye(ah, dtype=jnp.float32), (kk, 1))

  W1n = W1[:d_feat]
  W1g = W1[d_feat:d_feat + d_glob]
  W1a = W1[d_feat + d_glob:d_feat + d_glob + ah]
  W1h = W1[d_feat + d_glob + ah:]
  b1_eff = (b1 + (globals_ @ W1g)[0])[None, :]

  idx = edge_ind.astype(jnp.int32).reshape(-1)
  gathered2 = _sc_gather(edges, idx).reshape(n, kk * d_edge)

  seg = hyper_ind.astype(jnp.int32)
  hyp_parts = _sc_segsum(hyper_feat, seg, n).reshape(2, n, d_hyp)

  out = _tc_main(nodes, gathered2, hyp_parts, Wq_t, bq_t, W_bd, bk_t, S,
                 W1n, W1a, W1h, b1_eff, W2, b2[None, :],
                 gamma[None, :], beta[None, :])
  return out


# speedup vs baseline: 1.4049x; 1.1478x over previous
import functools

import jax
import jax.numpy as jnp
from jax import lax
from jax.experimental import pallas as pl
from jax.experimental.pallas import tpu as pltpu
from jax.experimental.pallas import tpu_sc as plsc


def _sc_gather(table, idx, chunk=2000):
  B = idx.shape[0]
  D = table.shape[1]
  info = plsc.get_sparse_core_info()
  nw = info.num_cores * info.num_subcores
  b_per_w = B // nw
  assert B % nw == 0 and b_per_w % chunk == 0, (B, nw, chunk)
  n_iter = b_per_w // chunk
  mesh = plsc.VectorSubcoreMesh(core_axis_name="c", subcore_axis_name="s")

  @functools.partial(
      pl.kernel,
      out_type=jax.ShapeDtypeStruct((B, D), table.dtype),
      mesh=mesh,
      scratch_types=[
          pltpu.VMEM((chunk,), jnp.int32),
          pltpu.VMEM((chunk, D), table.dtype),
          pltpu.SemaphoreType.DMA,
      ],
      compiler_params=pltpu.CompilerParams(use_tc_tiling_on_sc=False),
  )
  def k(table_hbm, idx_hbm, out_hbm, idx_v, rows_v, sem):
    wid = lax.axis_index("s") * info.num_cores + lax.axis_index("c")
    base = wid * b_per_w

    def body(i, _):
      start = base + i * chunk
      pltpu.sync_copy(idx_hbm.at[pl.ds(start, chunk)], idx_v)
      pltpu.async_copy(table_hbm.at[idx_v], rows_v, sem).wait()
      pltpu.sync_copy(rows_v, out_hbm.at[pl.ds(start, chunk)])
      return 0

    lax.fori_loop(0, n_iter, body, 0)

  return k(table, idx)


def _phys4(x):
  r, d = x.shape
  assert d == 16 and r % 128 == 0
  return x.T.reshape(2, 8, r // 128, 128).swapaxes(1, 2)


def _sc_segsum(data, seg, n_out, nt_chunk=4, zchunk=400):
  H, D = data.shape
  data4 = _phys4(data)
  nt_all = H // 128
  info = plsc.get_sparse_core_info()
  nc, ns = info.num_cores, info.num_subcores
  nw = nc * ns
  assert nt_all % nt_chunk == 0
  n_chunks = nt_all // nt_chunk
  chunk = nt_chunk * 128
  assert n_out % zchunk == 0 and zchunk % 8 == 0 and zchunk <= chunk
  n_z = n_out // zchunk
  mesh = plsc.VectorSubcoreMesh(core_axis_name="c", subcore_axis_name="s")

  @functools.partial(
      pl.kernel,
      out_type=jax.ShapeDtypeStruct((nc * n_out, D), jnp.float32),
      mesh=mesh,
      scratch_types=[
          pltpu.VMEM((chunk,), jnp.int32),
          pltpu.VMEM((D * chunk,), jnp.float32),
          pltpu.VMEM((chunk, D), jnp.float32),
          pltpu.VMEM_SHARED((n_out, D), jnp.float32),
          pltpu.SemaphoreType.DMA,
      ],
      compiler_params=pltpu.CompilerParams(use_tc_tiling_on_sc=False,
                                           needs_layout_passes=False),
  )
  def k(data_hbm, seg_hbm, out_hbm, idx_v, stage_v, rows_v, acc_sp, sem):
    cid = lax.axis_index("c")
    sid = lax.axis_index("s")
    wid = sid * nc + cid

    def zrow(i, _):
      rows_v[i] = jnp.zeros((D,), jnp.float32)
      return 0

    lax.fori_loop(0, chunk, zrow, 0)

    def zbody(c, _):
      pltpu.sync_copy(rows_v.at[pl.ds(0, zchunk)],
                      acc_sp.at[pl.ds(c * zchunk, zchunk)])
      return 0

    def zloop(t0):
      n_mine = (n_z - t0 + ns - 1) // ns
      lax.fori_loop(0, n_mine, lambda j, _: zbody(t0 + j * ns, _), 0)

    zloop(sid)
    plsc.subcore_barrier()

    n_mine = (n_chunks - wid + nw - 1) // nw
    base_vec = lax.iota(jnp.int32, 16) * chunk
    offs = [jnp.full((16,), u, jnp.int32) for u in range(8)]
    eight = jnp.full((16,), 8, jnp.int32)

    def body(i, _):
      cidx = wid + i * nw
      t0 = cidx * nt_chunk
      pltpu.sync_copy(seg_hbm.at[pl.ds(t0 * 128, chunk)], idx_v)
      cps = []
      for tr in range(2):
        for r in range(8):
          c = tr * 8 + r
          for t in range(nt_chunk):
            cps.append(pltpu.async_copy(
                data_hbm.at[tr, t0 + t, r, :],
                stage_v.at[pl.ds(c * chunk + t * 128, 128)], sem))
      for cp in cps:
        cp.wait()

      def asm8(g, vec):
        for u in range(8):
          rows_v[g * 8 + u] = plsc.load_gather(stage_v, [vec + offs[u]])
        return vec + eight

      lax.fori_loop(0, chunk // 8, asm8, base_vec)
      pltpu.sync_copy(rows_v, acc_sp.at[idx_v], add=True)
      return 0

    lax.fori_loop(0, n_mine, body, 0)
    plsc.subcore_barrier()

    def wbody(c, _):
      o = c * zchunk
      pltpu.sync_copy(acc_sp.at[pl.ds(o, zchunk)],
                      out_hbm.at[pl.ds(cid * n_out + o, zchunk)])
      return 0

    def wloop(t0):
      n_mine = (n_z - t0 + ns - 1) // ns
      lax.fori_loop(0, n_mine, lambda j, _: wbody(t0 + j * ns, _), 0)

    wloop(sid)

  return k(data4, seg)


def _tc_main(nodes, gathered2, hyp_parts, Wq_t, bq_t, W_bd, bk_t, S,
             W1n, W1a, W1h, b1_eff, W2, b2, gamma, beta, nb=1000):
  n = nodes.shape[0]
  d_h = hyp_parts.shape[2]
  l2 = W2.shape[1]
  assert n % nb == 0

  def body(x_ref, g_ref, hp_ref, wqt_ref, bqt_ref, wbd_ref, bkt_ref, s_ref,
           w1n_ref, w1a_ref, w1h_ref, b1_ref, w2_ref, b2_ref,
           gamma_ref, beta_ref, o_ref):
    f32 = jnp.float32
    x = x_ref[...]
    qh_t = jnp.dot(x, wqt_ref[...], preferred_element_type=f32) + bqt_ref[...]
    kg = jnp.dot(g_ref[...], wbd_ref[...], preferred_element_type=f32)
    kg = kg + bkt_ref[...]
    s = qh_t * kg
    m = jnp.max(s, axis=-1, keepdims=True)
    w = jnp.exp(s - m)
    sel = s_ref[...]
    z = jnp.dot(w, sel, preferred_element_type=f32)
    att = jnp.dot(w * kg, sel, preferred_element_type=f32) / z
    hyp = hp_ref[0] + hp_ref[1]
    pre1 = (jnp.dot(x, w1n_ref[...], preferred_element_type=f32)
            + jnp.dot(att, w1a_ref[...], preferred_element_type=f32)
            + jnp.dot(hyp, w1h_ref[...], preferred_element_type=f32)
            + b1_ref[...])
    h1 = jnp.maximum(pre1, 0.0)
    h2 = jnp.dot(h1, w2_ref[...], preferred_element_type=f32)
    h2 = jnp.maximum(h2 + b2_ref[...], 0.0)
    mean = jnp.mean(h2, axis=-1, keepdims=True)
    var = jnp.mean((h2 - mean) * (h2 - mean), axis=-1, keepdims=True)
    o_ref[...] = ((h2 - mean) * lax.rsqrt(var + 1e-3) * gamma_ref[...]
                  + beta_ref[...])

  grid = (n // nb,)
  full = lambda shape: pl.BlockSpec(shape, lambda i: (0,) * len(shape))
  return pl.pallas_call(
      body,
      grid=grid,
      in_specs=[
          pl.BlockSpec((nb, nodes.shape[1]), lambda i: (i, 0)),
          pl.BlockSpec((nb, gathered2.shape[1]), lambda i: (i, 0)),
          pl.BlockSpec((2, nb, d_h), lambda i: (0, i, 0)),
          full(Wq_t.shape), full(bq_t.shape), full(W_bd.shape), full(bk_t.shape),
          full(S.shape),
          full(W1n.shape), full(W1a.shape), full(W1h.shape), full(b1_eff.shape),
          full(W2.shape), full(b2.shape), full(gamma.shape), full(beta.shape),
      ],
      out_specs=pl.BlockSpec((nb, l2), lambda i: (i, 0)),
      out_shape=jax.ShapeDtypeStruct((n, l2), jnp.float32),
  )(nodes, gathered2, hyp_parts, Wq_t, bq_t, W_bd, bk_t, S,
    W1n, W1a, W1h, b1_eff, W2, b2, gamma, beta)


def kernel(nodes, globals_, edges, edge_ind, hyper_feat, hyper_ind,
           Wq, bq, Wk, bk, Wc, bc, W1, b1, W2, b2, gamma, beta):
  n, d_feat = nodes.shape
  e, d_edge = edges.shape
  kk = edge_ind.shape[1]
  d_glob = globals_.shape[1]
  ah = Wc.shape[2]
  d_hyp = hyper_feat.shape[1]

  Wc1 = Wc[1]
  Wqc = Wq @ Wc1
  bqc = (bq @ Wc1 + bc)[None, :]
  Wkc = Wk @ Wc1
  bkc = (bk @ Wc1 + bc)[None, :]

  Wq_t = jnp.tile(Wqc, (1, kk))
  bq_t = jnp.tile(bqc, (1, kk))
  W_bd = jnp.kron(jnp.eye(kk, dtype=jnp.float32), Wkc)
  bk_t = jnp.tile(bkc, (1, kk))
  S = jnp.tile(jnp.eye(ah, dtype=jnp.float32), (kk, 1))

  W1n = W1[:d_feat]
  W1g = W1[d_feat:d_feat + d_glob]
  W1a = W1[d_feat + d_glob:d_feat + d_glob + ah]
  W1h = W1[d_feat + d_glob + ah:]
  b1_eff = (b1 + (globals_ @ W1g)[0])[None, :]

  idx = edge_ind.astype(jnp.int32).reshape(-1)
  gathered2 = _sc_gather(edges, idx).reshape(n, kk * d_edge)

  seg = hyper_ind.astype(jnp.int32)
  hyp_parts = _sc_segsum(hyper_feat, seg, n).reshape(2, n, d_hyp)

  out = _tc_main(nodes, gathered2, hyp_parts, Wq_t, bq_t, W_bd, bk_t, S,
                 W1n, W1a, W1h, b1_eff, W2, b2[None, :],
                 gamma[None, :], beta[None, :])
  return out
